# baseline (speedup 1.0000x reference)
import functools

import jax
import jax.numpy as jnp
from jax import lax
from jax.experimental import pallas as pl
from jax.experimental.pallas import tpu as pltpu
from jax.experimental.pallas import tpu_sc as plsc

_VOCAB = 1_000_000
_DIM = 32
_BATCH = 16384

_NC = 2
_NS = 16
_L = 16
_NW = _NC * _NS
_BPW = _BATCH // _NW
_B = 4
_NBLK = _BPW // _B
_TAIL0 = (_VOCAB // 128) * 128
_NTAIL = _VOCAB - _TAIL0
_WROWS = _B * _DIM

_mesh = plsc.VectorSubcoreMesh(core_axis_name="c", subcore_axis_name="s")


@functools.partial(
    pl.kernel,
    mesh=_mesh,
    out_type=jax.ShapeDtypeStruct((_BATCH,), jnp.float32),
    compiler_params=pltpu.CompilerParams(needs_layout_passes=False),
    scratch_types=[
        pltpu.VMEM((_BPW,), jnp.int32),
        pltpu.VMEM((_BPW,), jnp.int32),
        pltpu.VMEM((2 * _WROWS, 128), jnp.float32),
        pltpu.VMEM((2 * _WROWS, 128), jnp.float32),
        pltpu.VMEM((2 * _B * _DIM,), jnp.float32),
        pltpu.VMEM((2 * _B * _DIM,), jnp.float32),
        pltpu.VMEM((_BPW,), jnp.float32),
        pltpu.SemaphoreType.DMA,
        pltpu.SemaphoreType.DMA,
    ],
)
def _w2v_kernel(w1_hbm, w2_hbm, embt_hbm, tail_hbm, out_hbm,
                idx1_v, idx2_v, w1buf, w2buf, t1buf, t2buf, out_v,
                sem1, sem2):
    wid = lax.axis_index("s") * _NC + lax.axis_index("c")
    base = wid * _BPW

    pltpu.sync_copy(w1_hbm.at[wid], idx1_v)
    pltpu.sync_copy(w2_hbm.at[wid], idx2_v)

    iota16 = lax.iota(jnp.int32, _L)
    maxq = (_TAIL0 // 128) - 1

    def issue_block(g, slot):
        iv1 = idx1_v[pl.ds(g * _B, _L)]
        iv2 = idx2_v[pl.ds(g * _B, _L)]
        q1 = jnp.minimum(iv1 >> 7, maxq) * 128
        q2 = jnp.minimum(iv2 >> 7, maxq) * 128
        t1 = jnp.clip(iv1 - _TAIL0, 0, _NTAIL - 1) * _DIM
        t2 = jnp.clip(iv2 - _TAIL0, 0, _NTAIL - 1) * _DIM
        for s in range(_B):
            for r in range(4):
                row = slot * _WROWS + s * _DIM + r * 8
                pltpu.make_async_copy(
                    embt_hbm.at[pl.ds(r * 8, 8),
                                pl.ds(pl.multiple_of(q1[s], 128), 128)],
                    w1buf.at[pl.ds(row, 8), :], sem1).start()
                pltpu.make_async_copy(
                    embt_hbm.at[pl.ds(r * 8, 8),
                                pl.ds(pl.multiple_of(q2[s], 128), 128)],
                    w2buf.at[pl.ds(row, 8), :], sem2).start()
            trow = (slot * _B + s) * _DIM
            pltpu.make_async_copy(
                tail_hbm.at[pl.ds(pl.multiple_of(t1[s], _DIM), _DIM)],
                t1buf.at[pl.ds(trow, _DIM)], sem1).start()
            pltpu.make_async_copy(
                tail_hbm.at[pl.ds(pl.multiple_of(t2[s], _DIM), _DIM)],
                t2buf.at[pl.ds(trow, _DIM)], sem2).start()

    def drain_block(slot):
        rows = pl.ds(slot * _WROWS, _WROWS)
        tsl = pl.ds(slot * _B * _DIM, _B * _DIM)
        pltpu.make_async_copy(embt_hbm.at[pl.ds(0, _WROWS), pl.ds(0, 128)],
                              w1buf.at[rows, :], sem1).wait()
        pltpu.make_async_copy(tail_hbm.at[pl.ds(0, _B * _DIM)],
                              t1buf.at[tsl], sem1).wait()
        pltpu.make_async_copy(embt_hbm.at[pl.ds(0, _WROWS), pl.ds(0, 128)],
                              w2buf.at[rows, :], sem2).wait()
        pltpu.make_async_copy(tail_hbm.at[pl.ds(0, _B * _DIM)],
                              t2buf.at[tsl], sem2).wait()

    def extract_block(g, slot, acc):
        iv1 = idx1_v[pl.ds(g * _B, _L)]
        iv2 = idx2_v[pl.ds(g * _B, _L)]
        q1 = jnp.minimum(iv1 >> 7, maxq) * 128
        q2 = jnp.minimum(iv2 >> 7, maxq) * 128
        col1 = iv1 - q1
        col2 = iv2 - q2
        for s in range(_B):
            dvec = iota16
            wrow_lo = slot * _WROWS + s * _DIM + dvec
            wrow_hi = wrow_lo + _L
            c1 = jnp.full((_L,), 1, jnp.int32) * col1[s]
            c2 = jnp.full((_L,), 1, jnp.int32) * col2[s]
            a_lo = plsc.load_gather(w1buf, [wrow_lo, c1])
            a_hi = plsc.load_gather(w1buf, [wrow_hi, c1])
            b_lo = plsc.load_gather(w2buf, [wrow_lo, c2])
            b_hi = plsc.load_gather(w2buf, [wrow_hi, c2])
            trow = (slot * _B + s) * _DIM
            at_lo = t1buf[pl.ds(trow, _L)]
            at_hi = t1buf[pl.ds(trow + _L, _L)]
            bt_lo = t2buf[pl.ds(trow, _L)]
            bt_hi = t2buf[pl.ds(trow + _L, _L)]
            is_t1 = iv1[s] >= _TAIL0
            is_t2 = iv2[s] >= _TAIL0
            a_lo = jnp.where(is_t1, at_lo, a_lo)
            a_hi = jnp.where(is_t1, at_hi, a_hi)
            b_lo = jnp.where(is_t2, bt_lo, b_lo)
            b_hi = jnp.where(is_t2, bt_hi, b_hi)
            dot = jnp.sum(a_lo * b_lo + a_hi * b_hi)
            lane = (g * _B + s) % _L
            acc = jnp.where(iota16 == lane, dot, acc)
        return acc

    issue_block(0, 0)

    def body(g, acc):
        slot = g & 1

        @pl.when(g + 1 < _NBLK)
        def _():
            issue_block(g + 1, 1 - slot)

        drain_block(slot)
        acc = extract_block(g, slot, acc)

        @pl.when((g & 3) == 3)
        def _():
            out_v[pl.ds((g - 3) * _B, _L)] = 1.0 / (1.0 + jnp.exp(-acc))

        return acc

    lax.fori_loop(0, _NBLK, body, jnp.zeros((_L,), jnp.float32))

    pltpu.sync_copy(out_v, out_hbm.at[pl.ds(base, _BPW)])


def kernel(word1, word2, emb):
    w1 = word1.astype(jnp.int32).reshape(_NW, _BPW)
    w2 = word2.astype(jnp.int32).reshape(_NW, _BPW)
    embt = emb.T
    tail = emb[_TAIL0:, :].reshape(_NTAIL * _DIM)
    return _w2v_kernel(w1, w2, embt, tail)


# speedup vs baseline: 2.4349x; 2.4349x over previous
import functools

import jax
import jax.numpy as jnp
from jax import lax
from jax.experimental import pallas as pl
from jax.experimental.pallas import tpu as pltpu
from jax.experimental.pallas import tpu_sc as plsc

_VOCAB = 1_000_000
_DIM = 32
_BATCH = 16384

_NC = 2
_NS = 16
_L = 16
_NW = _NC * _NS
_BPW = _BATCH // _NW
_B = 4
_NBLK = _BPW // _B
_TAIL0 = (_VOCAB // 128) * 128
_NTAIL = _VOCAB - _TAIL0
_WROWS = _B * _DIM

_mesh = plsc.VectorSubcoreMesh(core_axis_name="c", subcore_axis_name="s")


@functools.partial(
    pl.kernel,
    mesh=_mesh,
    out_type=jax.ShapeDtypeStruct((_BATCH,), jnp.float32),
    compiler_params=pltpu.CompilerParams(needs_layout_passes=False),
    scratch_types=[
        pltpu.VMEM((_BPW,), jnp.int32),
        pltpu.VMEM((_BPW,), jnp.int32),
        pltpu.VMEM((2 * _WROWS, 128), jnp.float32),
        pltpu.VMEM((2 * _WROWS, 128), jnp.float32),
        pltpu.VMEM((_NTAIL * _DIM,), jnp.float32),
        pltpu.VMEM((_BPW,), jnp.float32),
        pltpu.SemaphoreType.DMA,
        pltpu.SemaphoreType.DMA,
    ],
)
def _w2v_kernel(w1_hbm, w2_hbm, embt_hbm, tail_hbm, out_hbm,
                idx1_v, idx2_v, w1buf, w2buf, tbuf, out_v,
                sem1, sem2):
    wid = lax.axis_index("s") * _NC + lax.axis_index("c")
    base = wid * _BPW

    pltpu.sync_copy(w1_hbm.at[wid], idx1_v)
    pltpu.sync_copy(w2_hbm.at[wid], idx2_v)
    pltpu.sync_copy(tail_hbm, tbuf)

    iota16 = lax.iota(jnp.int32, _L)
    maxq = (_TAIL0 // 128) - 1

    def issue_block(g, slot):
        iv1 = idx1_v[pl.ds(g * _B, _L)]
        iv2 = idx2_v[pl.ds(g * _B, _L)]
        q1 = jnp.minimum(iv1 >> 7, maxq) * 128
        q2 = jnp.minimum(iv2 >> 7, maxq) * 128
        for s in range(_B):
            row = slot * _WROWS + s * _DIM
            pltpu.make_async_copy(
                embt_hbm.at[pl.ds(0, _DIM),
                            pl.ds(pl.multiple_of(q1[s], 128), 128)],
                w1buf.at[pl.ds(row, _DIM), :], sem1).start()
            pltpu.make_async_copy(
                embt_hbm.at[pl.ds(0, _DIM),
                            pl.ds(pl.multiple_of(q2[s], 128), 128)],
                w2buf.at[pl.ds(row, _DIM), :], sem2).start()

    def drain_block(slot):
        rows = pl.ds(slot * _WROWS, _WROWS)
        pltpu.make_async_copy(embt_hbm.at[pl.ds(0, _WROWS), pl.ds(0, 128)],
                              w1buf.at[rows, :], sem1).wait()
        pltpu.make_async_copy(embt_hbm.at[pl.ds(0, _WROWS), pl.ds(0, 128)],
                              w2buf.at[rows, :], sem2).wait()

    def extract_block(g, slot, acc):
        iv1 = idx1_v[pl.ds(g * _B, _L)]
        iv2 = idx2_v[pl.ds(g * _B, _L)]
        q1 = jnp.minimum(iv1 >> 7, maxq) * 128
        q2 = jnp.minimum(iv2 >> 7, maxq) * 128
        col1 = iv1 - q1
        col2 = iv2 - q2
        for s in range(_B):
            dvec = iota16
            wrow_lo = slot * _WROWS + s * _DIM + dvec
            wrow_hi = wrow_lo + _L
            c1 = jnp.full((_L,), 1, jnp.int32) * col1[s]
            c2 = jnp.full((_L,), 1, jnp.int32) * col2[s]
            a_lo = plsc.load_gather(w1buf, [wrow_lo, c1])
            a_hi = plsc.load_gather(w1buf, [wrow_hi, c1])
            b_lo = plsc.load_gather(w2buf, [wrow_lo, c2])
            b_hi = plsc.load_gather(w2buf, [wrow_hi, c2])
            tr1 = jnp.clip(iv1[s] - _TAIL0, 0, _NTAIL - 1) * _DIM
            tr2 = jnp.clip(iv2[s] - _TAIL0, 0, _NTAIL - 1) * _DIM
            at_lo = plsc.load_gather(tbuf, [tr1 + dvec])
            at_hi = plsc.load_gather(tbuf, [tr1 + _L + dvec])
            bt_lo = plsc.load_gather(tbuf, [tr2 + dvec])
            bt_hi = plsc.load_gather(tbuf, [tr2 + _L + dvec])
            is_t1 = iv1[s] >= _TAIL0
            is_t2 = iv2[s] >= _TAIL0
            a_lo = jnp.where(is_t1, at_lo, a_lo)
            a_hi = jnp.where(is_t1, at_hi, a_hi)
            b_lo = jnp.where(is_t2, bt_lo, b_lo)
            b_hi = jnp.where(is_t2, bt_hi, b_hi)
            dot = jnp.sum(a_lo * b_lo + a_hi * b_hi)
            lane = (g * _B + s) % _L
            acc = jnp.where(iota16 == lane, dot, acc)
        return acc

    issue_block(0, 0)

    def body(g, acc):
        slot = g & 1

        @pl.when(g + 1 < _NBLK)
        def _():
            issue_block(g + 1, 1 - slot)

        drain_block(slot)
        acc = extract_block(g, slot, acc)

        @pl.when((g & 3) == 3)
        def _():
            out_v[pl.ds((g - 3) * _B, _L)] = 1.0 / (1.0 + jnp.exp(-acc))

        return acc

    lax.fori_loop(0, _NBLK, body, jnp.zeros((_L,), jnp.float32))

    pltpu.sync_copy(out_v, out_hbm.at[pl.ds(base, _BPW)])


def kernel(word1, word2, emb):
    w1 = word1.astype(jnp.int32).reshape(_NW, _BPW)
    w2 = word2.astype(jnp.int32).reshape(_NW, _BPW)
    embt = emb.T
    tail = emb[_TAIL0:, :].reshape(_NTAIL * _DIM)
    return _w2v_kernel(w1, w2, embt, tail)


# speedup vs baseline: 2.6763x; 1.0991x over previous
import functools

import jax
import jax.numpy as jnp
from jax import lax
from jax.experimental import pallas as pl
from jax.experimental.pallas import tpu as pltpu
from jax.experimental.pallas import tpu_sc as plsc

_VOCAB = 1_000_000
_DIM = 32
_BATCH = 16384

_NC = 2
_NS = 16
_L = 16
_NW = _NC * _NS
_BPW = _BATCH // _NW
_B = 4
_NBLK = _BPW // _B
_TAIL0 = (_VOCAB // 128) * 128
_NTAIL = _VOCAB - _TAIL0
_WROWS = _B * _DIM

_mesh = plsc.VectorSubcoreMesh(core_axis_name="c", subcore_axis_name="s")


@functools.partial(
    pl.kernel,
    mesh=_mesh,
    out_type=jax.ShapeDtypeStruct((_BATCH,), jnp.float32),
    compiler_params=pltpu.CompilerParams(needs_layout_passes=False),
    scratch_types=[
        pltpu.VMEM((_BPW,), jnp.int32),
        pltpu.VMEM((_BPW,), jnp.int32),
        pltpu.VMEM((3 * _WROWS, 128), jnp.float32),
        pltpu.VMEM((3 * _WROWS, 128), jnp.float32),
        pltpu.VMEM((_NTAIL * _DIM,), jnp.float32),
        pltpu.VMEM((_BPW,), jnp.float32),
        pltpu.SemaphoreType.DMA,
        pltpu.SemaphoreType.DMA,
    ],
)
def _w2v_kernel(w1_hbm, w2_hbm, embt_hbm, tail_hbm, out_hbm,
                idx1_v, idx2_v, w1buf, w2buf, tbuf, out_v,
                sem1, sem2):
    wid = lax.axis_index("s") * _NC + lax.axis_index("c")
    base = wid * _BPW

    pltpu.sync_copy(w1_hbm.at[wid], idx1_v)
    pltpu.sync_copy(w2_hbm.at[wid], idx2_v)
    pltpu.sync_copy(tail_hbm, tbuf)

    iota16 = lax.iota(jnp.int32, _L)
    maxq = (_TAIL0 // 128) - 1

    def issue_block(g, slot):
        iv1 = idx1_v[pl.ds(g * _B, _L)]
        iv2 = idx2_v[pl.ds(g * _B, _L)]
        q1 = jnp.minimum(iv1 >> 7, maxq) * 128
        q2 = jnp.minimum(iv2 >> 7, maxq) * 128
        for s in range(_B):
            row = slot * _WROWS + s * _DIM
            pltpu.make_async_copy(
                embt_hbm.at[pl.ds(0, _DIM),
                            pl.ds(pl.multiple_of(q1[s], 128), 128)],
                w1buf.at[pl.ds(row, _DIM), :], sem1).start()
            pltpu.make_async_copy(
                embt_hbm.at[pl.ds(0, _DIM),
                            pl.ds(pl.multiple_of(q2[s], 128), 128)],
                w2buf.at[pl.ds(row, _DIM), :], sem2).start()

    def drain_block(slot):
        rows = pl.ds(slot * _WROWS, _WROWS)
        pltpu.make_async_copy(embt_hbm.at[pl.ds(0, _WROWS), pl.ds(0, 128)],
                              w1buf.at[rows, :], sem1).wait()
        pltpu.make_async_copy(embt_hbm.at[pl.ds(0, _WROWS), pl.ds(0, 128)],
                              w2buf.at[rows, :], sem2).wait()

    def extract_block(g, slot, acc):
        iv1 = idx1_v[pl.ds(g * _B, _L)]
        iv2 = idx2_v[pl.ds(g * _B, _L)]
        q1 = jnp.minimum(iv1 >> 7, maxq) * 128
        q2 = jnp.minimum(iv2 >> 7, maxq) * 128
        col1 = iv1 - q1
        col2 = iv2 - q2
        for s in range(_B):
            dvec = iota16
            wrow_lo = slot * _WROWS + s * _DIM + dvec
            wrow_hi = wrow_lo + _L
            c1 = jnp.full((_L,), 1, jnp.int32) * col1[s]
            c2 = jnp.full((_L,), 1, jnp.int32) * col2[s]
            a_lo = plsc.load_gather(w1buf, [wrow_lo, c1])
            a_hi = plsc.load_gather(w1buf, [wrow_hi, c1])
            b_lo = plsc.load_gather(w2buf, [wrow_lo, c2])
            b_hi = plsc.load_gather(w2buf, [wrow_hi, c2])
            tr1 = jnp.clip(iv1[s] - _TAIL0, 0, _NTAIL - 1) * _DIM
            tr2 = jnp.clip(iv2[s] - _TAIL0, 0, _NTAIL - 1) * _DIM
            at_lo = plsc.load_gather(tbuf, [tr1 + dvec])
            at_hi = plsc.load_gather(tbuf, [tr1 + _L + dvec])
            bt_lo = plsc.load_gather(tbuf, [tr2 + dvec])
            bt_hi = plsc.load_gather(tbuf, [tr2 + _L + dvec])
            is_t1 = iv1[s] >= _TAIL0
            is_t2 = iv2[s] >= _TAIL0
            a_lo = jnp.where(is_t1, at_lo, a_lo)
            a_hi = jnp.where(is_t1, at_hi, a_hi)
            b_lo = jnp.where(is_t2, bt_lo, b_lo)
            b_hi = jnp.where(is_t2, bt_hi, b_hi)
            dot = jnp.sum(a_lo * b_lo + a_hi * b_hi)
            lane = (g * _B + s) % _L
            acc = jnp.where(iota16 == lane, dot, acc)
        return acc

    issue_block(0, 0)
    issue_block(1, 1)

    def body(g, acc):
        slot = g % 3

        @pl.when(g + 2 < _NBLK)
        def _():
            issue_block(g + 2, (g + 2) % 3)

        drain_block(slot)
        acc = extract_block(g, slot, acc)

        @pl.when((g & 3) == 3)
        def _():
            out_v[pl.ds((g - 3) * _B, _L)] = 1.0 / (1.0 + jnp.exp(-acc))

        return acc

    lax.fori_loop(0, _NBLK, body, jnp.zeros((_L,), jnp.float32))

    pltpu.sync_copy(out_v, out_hbm.at[pl.ds(base, _BPW)])


def kernel(word1, word2, emb):
    w1 = word1.astype(jnp.int32).reshape(_NW, _BPW)
    w2 = word2.astype(jnp.int32).reshape(_NW, _BPW)
    embt = emb.T
    tail = emb[_TAIL0:, :].reshape(_NTAIL * _DIM)
    return _w2v_kernel(w1, w2, embt, tail)
